# Initial kernel scaffold; baseline (speedup 1.0000x reference)
import jax
import jax.numpy as jnp
from jax.experimental import pallas as pl


def kernel(x, edge_index, edge_weight, W1, b1, W2, b2, W3, b3, g1, be1, g2, be2, g3, be3, C1, cb1, C2, cb2, C3, cb3):
    raise NotImplementedError("write your pallas kernel here")


# speedup vs baseline: 5.2699x; 5.2699x over previous
import functools

import jax
import jax.numpy as jnp
from jax import lax
from jax.experimental import pallas as pl
from jax.experimental.pallas import tpu as pltpu
from jax.experimental.pallas import tpu_sc as plsc

_NC = 2
_NS = 16
_NW = _NC * _NS
_CH = 128
_ROWBLK = 1000


def _elu(h):
    return jnp.where(h > 0.0, h, jnp.exp(jnp.minimum(h, 0.0)) - 1.0)


def _gelu(x):
    return 0.5 * x * (1.0 + lax.erf(x * 0.7071067811865476))


def _mm_body(x_ref, w_ref, o_ref):
    o_ref[...] = jnp.dot(x_ref[...], w_ref[...],
                         preferred_element_type=jnp.float32)


def _tc_mm(x, w, out_rows):
    n, f = x.shape
    h = w.shape[1]
    return pl.pallas_call(
        _mm_body,
        grid=(n // _ROWBLK,),
        in_specs=[pl.BlockSpec((_ROWBLK, f), lambda i: (i, 0)),
                  pl.BlockSpec((f, h), lambda i: (0, 0))],
        out_specs=pl.BlockSpec((_ROWBLK, h), lambda i: (i, 0)),
        out_shape=jax.ShapeDtypeStruct((out_rows, h), jnp.float32),
    )(x, w)


def _combine_body(p_ref, b_ref, g_ref, be_ref, w_ref, o_ref):
    h = p_ref[0] + p_ref[1] + b_ref[...]
    h = _elu(h) * g_ref[...] + be_ref[...]
    o_ref[...] = jnp.dot(h, w_ref[...], preferred_element_type=jnp.float32)


def _tc_combine(p, b, g, be, w_next, n_valid):
    _, n, h = p.shape
    h2 = w_next.shape[1]
    return pl.pallas_call(
        _combine_body,
        grid=(n_valid // _ROWBLK,),
        in_specs=[pl.BlockSpec((2, _ROWBLK, h), lambda i: (0, i, 0)),
                  pl.BlockSpec((1, h), lambda i: (0, 0)),
                  pl.BlockSpec((1, h), lambda i: (0, 0)),
                  pl.BlockSpec((1, h), lambda i: (0, 0)),
                  pl.BlockSpec((h, h2), lambda i: (0, 0))],
        out_specs=pl.BlockSpec((_ROWBLK, h2), lambda i: (i, 0)),
        out_shape=jax.ShapeDtypeStruct((n, h2), jnp.float32),
    )(p, b.reshape(1, h), g.reshape(1, h), be.reshape(1, h), w_next)


def _head_body(p_ref, b_ref, g_ref, be_ref, c1_ref, cb1_ref, c2_ref, cb2_ref,
               c3_ref, cb3_ref, o_ref):
    h = p_ref[0] + p_ref[1] + b_ref[...]
    h = _elu(h) * g_ref[...] + be_ref[...]
    t = jnp.dot(h, c1_ref[...], preferred_element_type=jnp.float32) + cb1_ref[...]
    t = _gelu(t)
    t = jnp.dot(t, c2_ref[...], preferred_element_type=jnp.float32) + cb2_ref[...]
    t = _gelu(t)
    o_ref[...] = (jnp.dot(t, c3_ref[...], preferred_element_type=jnp.float32)
                  + cb3_ref[...])


def _tc_head(p, b, g, be, c1, cb1, c2, cb2, c3, cb3, n_valid):
    _, n, h = p.shape
    h1 = c1.shape[1]
    c = c3.shape[1]
    return pl.pallas_call(
        _head_body,
        grid=(n_valid // _ROWBLK,),
        in_specs=[pl.BlockSpec((2, _ROWBLK, h), lambda i: (0, i, 0)),
                  pl.BlockSpec((1, h), lambda i: (0, 0)),
                  pl.BlockSpec((1, h), lambda i: (0, 0)),
                  pl.BlockSpec((1, h), lambda i: (0, 0)),
                  pl.BlockSpec((h, h1), lambda i: (0, 0)),
                  pl.BlockSpec((1, h1), lambda i: (0, 0)),
                  pl.BlockSpec((h1, h), lambda i: (0, 0)),
                  pl.BlockSpec((1, h), lambda i: (0, 0)),
                  pl.BlockSpec((h, c), lambda i: (0, 0)),
                  pl.BlockSpec((1, c), lambda i: (0, 0))],
        out_specs=pl.BlockSpec((_ROWBLK, c), lambda i: (i, 0)),
        out_shape=jax.ShapeDtypeStruct((n_valid, c), jnp.float32),
    )(p, b.reshape(1, h), g.reshape(1, h), be.reshape(1, h),
      c1, cb1.reshape(1, h1), c2, cb2.reshape(1, h), c3, cb3.reshape(1, c))


def _sc_aggregate(hw, srcc, dstc, wc, zeros):
    n, h = hw.shape
    nch, ch = srcc.shape[1], srcc.shape[2]
    rpt = n // _NS
    nvec = h // 16

    @functools.partial(
        pl.kernel,
        out_type=jax.ShapeDtypeStruct((_NC, n, h), jnp.float32),
        mesh=plsc.VectorSubcoreMesh(core_axis_name="c", subcore_axis_name="s",
                                    num_cores=_NC, num_subcores=_NS),
        compiler_params=pltpu.CompilerParams(use_tc_tiling_on_sc=False),
        scratch_types=[
            pltpu.VMEM_SHARED((n, h), jnp.float32),
            pltpu.VMEM_SHARED((n, h), jnp.float32),
            pltpu.VMEM((nch, ch), jnp.int32),
            pltpu.VMEM((nch, ch), jnp.int32),
            pltpu.VMEM((nch, ch), jnp.float32),
            pltpu.VMEM((ch, h), jnp.float32),
            pltpu.VMEM((ch,), jnp.int32),
            pltpu.SemaphoreType.DMA,
        ],
    )
    def agg(hw_hbm, src_hbm, dst_hbm, w_hbm, z_hbm, out_hbm,
            hw_sh, acc_sh, src_v, dst_v, w_v, rows_v, dst_c, sem):
        c = lax.axis_index("c")
        s = lax.axis_index("s")
        wid = s * _NC + c
        r0 = s * rpt

        pltpu.sync_copy(hw_hbm.at[pl.ds(r0, rpt)], hw_sh.at[pl.ds(r0, rpt)])
        pltpu.sync_copy(z_hbm.at[pl.ds(r0, rpt)], acc_sh.at[pl.ds(r0, rpt)])
        pltpu.sync_copy(src_hbm.at[wid], src_v)
        pltpu.sync_copy(dst_hbm.at[wid], dst_v)
        pltpu.sync_copy(w_hbm.at[wid], w_v)
        plsc.subcore_barrier()

        def chunk_body(ci, carry):
            for g in range(ch // 16):
                sl = pl.ds(g * 16, 16)
                dst_c[sl] = dst_v[ci, sl]
            pltpu.async_copy(hw_sh.at[src_v.at[ci]], rows_v, sem).wait()

            def group_body(g, carry2):
                wv = w_v[ci, pl.ds(g * 16, 16)]
                for lane in range(16):
                    wsc = wv[lane]
                    e = g * 16 + lane
                    for j in range(nvec):
                        sl = pl.ds(j * 16, 16)
                        rows_v[e, sl] = rows_v[e, sl] * wsc
                return carry2

            lax.fori_loop(0, ch // 16, group_body, 0)
            pltpu.sync_copy(rows_v, acc_sh.at[dst_c], add=True)
            return carry

        lax.fori_loop(0, nch, chunk_body, 0)
        plsc.subcore_barrier()
        pltpu.sync_copy(acc_sh.at[pl.ds(r0, rpt)],
                        out_hbm.at[c, pl.ds(r0, rpt)])

    return agg(hw, srcc, dstc, wc, zeros)


def kernel(x, edge_index, edge_weight, W1, b1, W2, b2, W3, b3,
           g1, be1, g2, be2, g3, be3, C1, cb1, C2, cb2, C3, cb3):
    n = x.shape[0]
    e = edge_weight.shape[0]
    h = W1.shape[1]

    rpt = -(-n // (_NS * 8)) * 8
    n_pad = rpt * _NS
    per_w = _NW * _CH
    e_pad = -(-e // per_w) * per_w
    nch = e_pad // per_w
    pad = e_pad - e
    if pad:
        fill = (jnp.arange(pad, dtype=jnp.int32) * 8) % n
        src_full = jnp.concatenate([edge_index[0], fill])
        dst_full = jnp.concatenate([edge_index[1], fill])
        w_full = jnp.concatenate([edge_weight, jnp.zeros((pad,), jnp.float32)])
    else:
        src_full, dst_full, w_full = edge_index[0], edge_index[1], edge_weight
    srcc = src_full.reshape(_NW, nch, _CH)
    dstc = dst_full.reshape(_NW, nch, _CH)
    wc = w_full.reshape(_NW, nch, _CH)
    zeros = jnp.zeros((n_pad, h), jnp.float32)

    hw = _tc_mm(x, W1, n_pad)
    p = _sc_aggregate(hw, srcc, dstc, wc, zeros)
    hw = _tc_combine(p, b1, g1, be1, W2, n)
    p = _sc_aggregate(hw, srcc, dstc, wc, zeros)
    hw = _tc_combine(p, b2, g2, be2, W3, n)
    p = _sc_aggregate(hw, srcc, dstc, wc, zeros)
    return _tc_head(p, b3, g3, be3, C1, cb1, C2, cb2, C3, cb3, n)


# speedup vs baseline: 5.2731x; 1.0006x over previous
import functools

import jax
import jax.numpy as jnp
from jax import lax
from jax.experimental import pallas as pl
from jax.experimental.pallas import tpu as pltpu
from jax.experimental.pallas import tpu_sc as plsc

_NC = 2
_NS = 16
_NW = _NC * _NS
_CH = 128
_ROWBLK = 1000


def _elu(h):
    return jnp.where(h > 0.0, h, jnp.exp(jnp.minimum(h, 0.0)) - 1.0)


def _gelu(x):
    return 0.5 * x * (1.0 + lax.erf(x * 0.7071067811865476))


def _mm_body(x_ref, w_ref, o_ref):
    o_ref[...] = jnp.dot(x_ref[...], w_ref[...],
                         preferred_element_type=jnp.float32)


def _tc_mm(x, w, out_rows):
    n, f = x.shape
    h = w.shape[1]
    return pl.pallas_call(
        _mm_body,
        grid=(n // _ROWBLK,),
        in_specs=[pl.BlockSpec((_ROWBLK, f), lambda i: (i, 0)),
                  pl.BlockSpec((f, h), lambda i: (0, 0))],
        out_specs=pl.BlockSpec((_ROWBLK, h), lambda i: (i, 0)),
        out_shape=jax.ShapeDtypeStruct((out_rows, h), jnp.float32),
    )(x, w)


def _combine_body(p_ref, b_ref, g_ref, be_ref, w_ref, o_ref):
    h = p_ref[0] + p_ref[1] + b_ref[...]
    h = _elu(h) * g_ref[...] + be_ref[...]
    o_ref[...] = jnp.dot(h, w_ref[...], preferred_element_type=jnp.float32)


def _tc_combine(p, b, g, be, w_next, n_valid):
    _, n, h = p.shape
    h2 = w_next.shape[1]
    return pl.pallas_call(
        _combine_body,
        grid=(n_valid // _ROWBLK,),
        in_specs=[pl.BlockSpec((2, _ROWBLK, h), lambda i: (0, i, 0)),
                  pl.BlockSpec((1, h), lambda i: (0, 0)),
                  pl.BlockSpec((1, h), lambda i: (0, 0)),
                  pl.BlockSpec((1, h), lambda i: (0, 0)),
                  pl.BlockSpec((h, h2), lambda i: (0, 0))],
        out_specs=pl.BlockSpec((_ROWBLK, h2), lambda i: (i, 0)),
        out_shape=jax.ShapeDtypeStruct((n, h2), jnp.float32),
    )(p, b.reshape(1, h), g.reshape(1, h), be.reshape(1, h), w_next)


def _head_body(p_ref, b_ref, g_ref, be_ref, c1_ref, cb1_ref, c2_ref, cb2_ref,
               c3_ref, cb3_ref, o_ref):
    h = p_ref[0] + p_ref[1] + b_ref[...]
    h = _elu(h) * g_ref[...] + be_ref[...]
    t = jnp.dot(h, c1_ref[...], preferred_element_type=jnp.float32) + cb1_ref[...]
    t = _gelu(t)
    t = jnp.dot(t, c2_ref[...], preferred_element_type=jnp.float32) + cb2_ref[...]
    t = _gelu(t)
    o_ref[...] = (jnp.dot(t, c3_ref[...], preferred_element_type=jnp.float32)
                  + cb3_ref[...])


def _tc_head(p, b, g, be, c1, cb1, c2, cb2, c3, cb3, n_valid):
    _, n, h = p.shape
    h1 = c1.shape[1]
    c = c3.shape[1]
    return pl.pallas_call(
        _head_body,
        grid=(n_valid // _ROWBLK,),
        in_specs=[pl.BlockSpec((2, _ROWBLK, h), lambda i: (0, i, 0)),
                  pl.BlockSpec((1, h), lambda i: (0, 0)),
                  pl.BlockSpec((1, h), lambda i: (0, 0)),
                  pl.BlockSpec((1, h), lambda i: (0, 0)),
                  pl.BlockSpec((h, h1), lambda i: (0, 0)),
                  pl.BlockSpec((1, h1), lambda i: (0, 0)),
                  pl.BlockSpec((h1, h), lambda i: (0, 0)),
                  pl.BlockSpec((1, h), lambda i: (0, 0)),
                  pl.BlockSpec((h, c), lambda i: (0, 0)),
                  pl.BlockSpec((1, c), lambda i: (0, 0))],
        out_specs=pl.BlockSpec((_ROWBLK, c), lambda i: (i, 0)),
        out_shape=jax.ShapeDtypeStruct((n_valid, c), jnp.float32),
    )(p, b.reshape(1, h), g.reshape(1, h), be.reshape(1, h),
      c1, cb1.reshape(1, h1), c2, cb2.reshape(1, h), c3, cb3.reshape(1, c))


def _sc_aggregate(hw, srcc, dstc, wc, zeros):
    n, h = hw.shape
    nch, ch = srcc.shape[1], srcc.shape[2]
    rpt = n // _NS
    nvec = h // 16

    @functools.partial(
        pl.kernel,
        out_type=jax.ShapeDtypeStruct((_NC, n, h), jnp.float32),
        mesh=plsc.VectorSubcoreMesh(core_axis_name="c", subcore_axis_name="s",
                                    num_cores=_NC, num_subcores=_NS),
        compiler_params=pltpu.CompilerParams(use_tc_tiling_on_sc=False),
        scratch_types=[
            pltpu.VMEM_SHARED((n, h), jnp.float32),
            pltpu.VMEM_SHARED((n, h), jnp.float32),
            pltpu.VMEM((nch, ch), jnp.int32),
            pltpu.VMEM((nch, ch), jnp.int32),
            pltpu.VMEM((nch, ch), jnp.float32),
            pltpu.VMEM((ch, h), jnp.float32),
            pltpu.VMEM((ch,), jnp.int32),
            pltpu.SemaphoreType.DMA,
        ],
    )
    def agg(hw_hbm, src_hbm, dst_hbm, w_hbm, z_hbm, out_hbm,
            hw_sh, acc_sh, src_v, dst_v, w_v, rows_v, dst_c, sem):
        c = lax.axis_index("c")
        s = lax.axis_index("s")
        wid = s * _NC + c
        r0 = s * rpt

        pltpu.sync_copy(hw_hbm.at[pl.ds(r0, rpt)], hw_sh.at[pl.ds(r0, rpt)])
        pltpu.sync_copy(z_hbm.at[pl.ds(r0, rpt)], acc_sh.at[pl.ds(r0, rpt)])
        pltpu.sync_copy(src_hbm.at[wid], src_v)
        pltpu.sync_copy(dst_hbm.at[wid], dst_v)
        pltpu.sync_copy(w_hbm.at[wid], w_v)
        plsc.subcore_barrier()

        def chunk_body(ci, carry):
            for g in range(ch // 16):
                sl = pl.ds(g * 16, 16)
                dst_c[sl] = dst_v[ci, sl]
            pltpu.async_copy(hw_sh.at[src_v.at[ci]], rows_v, sem).wait()

            def group_body(gi, carry2):
                wv = w_v[ci, pl.ds(gi * 16, 16)]
                for lane in range(16):
                    wsc = wv[lane]
                    e = gi * 16 + lane
                    for j in range(nvec):
                        sl = pl.ds(j * 16, 16)
                        rows_v[e, sl] = rows_v[e, sl] * wsc
                return carry2

            lax.fori_loop(0, ch // 16, group_body, 0)
            pltpu.sync_copy(rows_v, acc_sh.at[dst_c], add=True)
            return carry

        lax.fori_loop(0, nch, chunk_body, 0)

        plsc.subcore_barrier()
        pltpu.sync_copy(acc_sh.at[pl.ds(r0, rpt)],
                        out_hbm.at[c, pl.ds(r0, rpt)])

    return agg(hw, srcc, dstc, wc, zeros)


def kernel(x, edge_index, edge_weight, W1, b1, W2, b2, W3, b3,
           g1, be1, g2, be2, g3, be3, C1, cb1, C2, cb2, C3, cb3):
    n = x.shape[0]
    e = edge_weight.shape[0]
    h = W1.shape[1]

    rpt = -(-n // (_NS * 8)) * 8
    n_pad = rpt * _NS
    per_w = _NW * _CH
    e_pad = -(-e // per_w) * per_w
    nch = e_pad // per_w
    pad = e_pad - e
    if pad:
        fill = (jnp.arange(pad, dtype=jnp.int32) * 8) % n
        src_full = jnp.concatenate([edge_index[0], fill])
        dst_full = jnp.concatenate([edge_index[1], fill])
        w_full = jnp.concatenate([edge_weight, jnp.zeros((pad,), jnp.float32)])
    else:
        src_full, dst_full, w_full = edge_index[0], edge_index[1], edge_weight
    srcc = src_full.reshape(_NW, nch, _CH)
    dstc = dst_full.reshape(_NW, nch, _CH)
    wc = w_full.reshape(_NW, nch, _CH)
    zeros = jnp.zeros((n_pad, h), jnp.float32)

    hw = _tc_mm(x, W1, n_pad)
    p = _sc_aggregate(hw, srcc, dstc, wc, zeros)
    hw = _tc_combine(p, b1, g1, be1, W2, n)
    p = _sc_aggregate(hw, srcc, dstc, wc, zeros)
    hw = _tc_combine(p, b2, g2, be2, W3, n)
    p = _sc_aggregate(hw, srcc, dstc, wc, zeros)
    return _tc_head(p, b3, g3, be3, C1, cb1, C2, cb2, C3, cb3, n)
